# Initial kernel scaffold; baseline (speedup 1.0000x reference)
import jax
import jax.numpy as jnp
from jax.experimental import pallas as pl


def kernel(atomic_numbers, pos, batch, edge_src, edge_dst, params):
    raise NotImplementedError("write your pallas kernel here")


# speedup vs baseline: 3.6346x; 3.6346x over previous
import functools

import numpy as np
import jax
import jax.numpy as jnp
from jax import lax
from jax.experimental import pallas as pl
from jax.experimental.pallas import tpu as pltpu
from jax.experimental.pallas import tpu_sc as plsc

_N = 10000
_E = 160000
_D = 480
_NB = 128
_NH = 4
_HD = _D // _NH
_MAXR = 5.0
_AVGDEG = 15.57930850982666
_NW = 32

_F32 = jnp.float32

_SEL = np.zeros((_D, 8), np.float32)
for _c in range(_D):
  _SEL[_c, _c // _HD] = 1.0
_SELT = np.ascontiguousarray(_SEL.T)


def _pad_rows(a, m):
  r = (-a.shape[0]) % m
  if r == 0:
    return a
  return jnp.concatenate([a, jnp.zeros((r,) + a.shape[1:], a.dtype)], axis=0)


def _pad2(a, rows, cols):
  return jnp.pad(a, ((0, rows - a.shape[0]), (0, cols - a.shape[1])))


def _sc_gather_call(table, idx, chunk):
  v_rows, d = table.shape
  b = idx.shape[0]
  bpw = b // _NW
  nch = bpw // chunk
  mesh = plsc.VectorSubcoreMesh(core_axis_name="c", subcore_axis_name="s")

  @functools.partial(
      pl.kernel,
      mesh=mesh,
      compiler_params=pltpu.CompilerParams(use_tc_tiling_on_sc=False),
      out_type=jax.ShapeDtypeStruct((b, d), _F32),
      scratch_types=[
          pltpu.VMEM((chunk,), jnp.int32),
          pltpu.VMEM((chunk, d), _F32),
          pltpu.SemaphoreType.DMA,
      ],
  )
  def gk(table_hbm, idx_hbm, out_hbm, idx_v, rows_v, sem):
    wid = lax.axis_index("s") * 2 + lax.axis_index("c")

    def body(j, carry):
      base = wid * bpw + j * chunk
      pltpu.sync_copy(idx_hbm.at[pl.ds(base, chunk)], idx_v)
      pltpu.async_copy(table_hbm.at[idx_v], rows_v, sem).wait()
      pltpu.sync_copy(rows_v, out_hbm.at[pl.ds(base, chunk)])
      return carry

    lax.fori_loop(0, nch, body, 0)

  return gk(table, idx)


def _gather_rows(table, idx, chunk=128):
  b0 = idx.shape[0]
  idxp = _pad_rows(idx.astype(jnp.int32), _NW * chunk)
  out = _sc_gather_call(table, idxp, chunk)
  return out[:b0]


def _mm(x, w, b=None, act=None, mul=None, residual=None):
  m, k = x.shape
  _, n = w.shape
  bm = 2000 if m % 2000 == 0 else 1000
  grid = m // bm

  has_b = b is not None
  has_mul = mul is not None
  has_res = residual is not None

  def body(*refs):
    i = 0
    x_ref = refs[i]; i += 1
    w_ref = refs[i]; i += 1
    acc = jnp.dot(x_ref[...], w_ref[...], preferred_element_type=_F32)
    if has_b:
      acc = acc + refs[i][...]; i += 1
    if act == "silu":
      acc = acc * jax.nn.sigmoid(acc)
    elif act == "gelu":
      acc = jax.nn.gelu(acc)
    if has_mul:
      acc = acc * refs[i][...]; i += 1
    if has_res:
      acc = acc + refs[i][...]; i += 1
    refs[i][...] = acc

  in_specs = [
      pl.BlockSpec((bm, k), lambda i: (i, 0)),
      pl.BlockSpec((k, n), lambda i: (0, 0)),
  ]
  args = [x, w]
  if has_b:
    in_specs.append(pl.BlockSpec((1, n), lambda i: (0, 0)))
    args.append(b.reshape(1, n))
  if has_mul:
    in_specs.append(pl.BlockSpec((bm, n), lambda i: (i, 0)))
    args.append(mul)
  if has_res:
    in_specs.append(pl.BlockSpec((bm, n), lambda i: (i, 0)))
    args.append(residual)

  return pl.pallas_call(
      body,
      grid=(grid,),
      in_specs=in_specs,
      out_specs=pl.BlockSpec((bm, n), lambda i: (i, 0)),
      out_shape=jax.ShapeDtypeStruct((m, n), _F32),
  )(*args)


def _ln(x, g, b):
  m, d = x.shape
  bm = 2000 if m % 2000 == 0 else 1000

  def body(x_ref, g_ref, b_ref, o_ref):
    xv = x_ref[...]
    mu = jnp.mean(xv, axis=1, keepdims=True)
    xc = xv - mu
    var = jnp.mean(xc * xc, axis=1, keepdims=True)
    o_ref[...] = xc * lax.rsqrt(var + 1e-5) * g_ref[...] + b_ref[...]

  return pl.pallas_call(
      body,
      grid=(m // bm,),
      in_specs=[
          pl.BlockSpec((bm, d), lambda i: (i, 0)),
          pl.BlockSpec((1, d), lambda i: (0, 0)),
          pl.BlockSpec((1, d), lambda i: (0, 0)),
      ],
      out_specs=pl.BlockSpec((bm, d), lambda i: (i, 0)),
      out_shape=jax.ShapeDtypeStruct((m, d), _F32),
  )(x, g.reshape(1, d), b.reshape(1, d))


def _edge_features(pos_src, pos_dst):
  e = pos_src.shape[0]
  bm = 2000

  def body(ps_ref, pd_ref, sh_ref, rbf_ref):
    dv = ps_ref[...] - pd_ref[...]
    d2 = jnp.sum(dv * dv, axis=1, keepdims=True)
    dlen = jnp.sqrt(d2)
    inv = 1.0 / jnp.maximum(dlen, 1e-9)
    lane = lax.broadcasted_iota(jnp.int32, (bm, 16), 1)
    xc = jnp.sum(jnp.where(lane == 0, dv, 0.0), axis=1, keepdims=True) * inv
    yc = jnp.sum(jnp.where(lane == 1, dv, 0.0), axis=1, keepdims=True) * inv
    zc = jnp.sum(jnp.where(lane == 2, dv, 0.0), axis=1, keepdims=True) * inv
    s3 = np.sqrt(3.0).astype(np.float32)
    s15 = np.sqrt(15.0).astype(np.float32)
    s5 = np.sqrt(5.0).astype(np.float32)
    vals = [
        jnp.ones_like(xc),
        s3 * xc, s3 * yc, s3 * zc,
        s15 * xc * yc, s15 * yc * zc,
        0.5 * s5 * (3.0 * zc * zc - 1.0),
        s15 * xc * zc,
        0.5 * s15 * (xc * xc - yc * yc),
    ]
    sh = jnp.zeros((bm, 16), _F32)
    for i, v in enumerate(vals):
      sh = jnp.where(lane == i, v, sh)
    sh_ref[...] = sh
    centers = lax.broadcasted_iota(jnp.int32, (bm, _NB), 1).astype(_F32) * (
        _MAXR / (_NB - 1.0))
    z = (dlen - centers) * (_NB / _MAXR)
    rbf_ref[...] = jnp.exp(-0.5 * z * z)

  return pl.pallas_call(
      body,
      grid=(e // bm,),
      in_specs=[
          pl.BlockSpec((bm, 16), lambda i: (i, 0)),
          pl.BlockSpec((bm, 16), lambda i: (i, 0)),
      ],
      out_specs=[
          pl.BlockSpec((bm, 16), lambda i: (i, 0)),
          pl.BlockSpec((bm, _NB), lambda i: (i, 0)),
      ],
      out_shape=[
          jax.ShapeDtypeStruct((e, 16), _F32),
          jax.ShapeDtypeStruct((e, _NB), _F32),
      ],
  )(pos_src, pos_dst)


def _edge_scores(q_e, k_e, gate8, extra8):
  e = q_e.shape[0]
  bm = 2000
  scale = 1.0 / np.sqrt(_HD)

  def body(q_ref, k_ref, g_ref, x_ref, s_ref, o_ref):
    prod = q_ref[...] * k_ref[...]
    s = jnp.dot(prod, s_ref[...], preferred_element_type=_F32) * scale
    o_ref[...] = s * g_ref[...] + x_ref[...]

  return pl.pallas_call(
      body,
      grid=(e // bm,),
      in_specs=[
          pl.BlockSpec((bm, _D), lambda i: (i, 0)),
          pl.BlockSpec((bm, _D), lambda i: (i, 0)),
          pl.BlockSpec((bm, 8), lambda i: (i, 0)),
          pl.BlockSpec((bm, 8), lambda i: (i, 0)),
          pl.BlockSpec((_D, 8), lambda i: (0, 0)),
      ],
      out_specs=pl.BlockSpec((bm, 8), lambda i: (i, 0)),
      out_shape=jax.ShapeDtypeStruct((e, 8), _F32),
  )(q_e, k_e, gate8, extra8, _SEL)


def _edge_exp_weight(score, smax_e, v_e):
  e = score.shape[0]
  bm = 2000

  def body(s_ref, m_ref, v_ref, t_ref, ex_ref, wv_ref):
    ex = jnp.exp(s_ref[...] - m_ref[...])
    ex_ref[...] = ex
    exb = jnp.dot(ex, t_ref[...], preferred_element_type=_F32)
    wv_ref[...] = v_ref[...] * exb

  return pl.pallas_call(
      body,
      grid=(e // bm,),
      in_specs=[
          pl.BlockSpec((bm, 8), lambda i: (i, 0)),
          pl.BlockSpec((bm, 8), lambda i: (i, 0)),
          pl.BlockSpec((bm, _D), lambda i: (i, 0)),
          pl.BlockSpec((8, _D), lambda i: (0, 0)),
      ],
      out_specs=[
          pl.BlockSpec((bm, 8), lambda i: (i, 0)),
          pl.BlockSpec((bm, _D), lambda i: (i, 0)),
      ],
      out_shape=[
          jax.ShapeDtypeStruct((e, 8), _F32),
          jax.ShapeDtypeStruct((e, _D), _F32),
      ],
  )(score, smax_e, v_e, _SELT)


def _agg_normalize(agg, denom):
  m = agg.shape[0]
  bm = 2000 if m % 2000 == 0 else 1000

  def body(a_ref, d_ref, t_ref, o_ref):
    rcp = 1.0 / (d_ref[...] + 1e-9)
    o_ref[...] = a_ref[...] * jnp.dot(rcp, t_ref[...],
                                      preferred_element_type=_F32)

  return pl.pallas_call(
      body,
      grid=(m // bm,),
      in_specs=[
          pl.BlockSpec((bm, _D), lambda i: (i, 0)),
          pl.BlockSpec((bm, 8), lambda i: (i, 0)),
          pl.BlockSpec((8, _D), lambda i: (0, 0)),
      ],
      out_specs=pl.BlockSpec((bm, _D), lambda i: (i, 0)),
      out_shape=jax.ShapeDtypeStruct((m, _D), _F32),
  )(agg, denom, _SELT)


def _forward_impl(atomic_numbers, pos, batch, edge_src, edge_dst, params):
  p = params
  del batch

  pos16 = jnp.pad(pos, ((0, 0), (0, 13)))
  ps = _gather_rows(pos16, edge_src, chunk=512)
  pd = _gather_rows(pos16, edge_dst, chunk=512)
  sh16, rbf = _edge_features(ps, pd)

  atom_emb = _gather_rows(p["atom_embed"], atomic_numbers, chunk=128)

  r1 = _mm(rbf, p["deg_w1"], p["deg_b1"], act="silu")
  r2 = _mm(r1, p["deg_w2"], p["deg_b2"], act="silu")
  w3p = _pad2(p["deg_w3"], 64, 16)
  b3p = jnp.pad(p["deg_b3"], (0, 7))
  msg16 = _mm(r2, w3p, b3p, mul=sh16)
  deg16 = jax.ops.segment_sum(msg16, edge_dst, num_segments=_N)
  projp = _pad2(p["deg_proj"], 16, _D) * np.float32(1.0 / np.sqrt(_AVGDEG))
  x = _mm(deg16, projp, residual=atom_emb)

  for i in range(4):
    pre = "blk%d_" % i
    h = _ln(x, p[pre + "ln1_g"], p[pre + "ln1_b"])
    qt = _mm(h, p[pre + "Wq"])
    kt = _mm(h, p[pre + "Wk"])
    vt = _mm(h, p[pre + "Wv"])
    q_e = _gather_rows(qt, edge_dst, chunk=128)
    k_e = _gather_rows(kt, edge_src, chunk=128)
    v_e = _gather_rows(vt, edge_src, chunk=128)

    g1 = _mm(rbf, p[pre + "gate_w1"], p[pre + "gate_b1"], act="silu")
    g2 = _mm(g1, p[pre + "gate_w2"], p[pre + "gate_b2"], act="silu")
    gw3p = _pad2(p[pre + "gate_w3"], 64, 8)
    gb3p = jnp.pad(p[pre + "gate_b3"], (0, 4))
    gate8 = _mm(g2, gw3p, gb3p)
    mixp = _pad2(p[pre + "sh_mix"], 16, 8)
    extra8 = _mm(sh16, mixp)

    score = _edge_scores(q_e, k_e, gate8, extra8)
    smax = jax.ops.segment_max(score, edge_dst, num_segments=_N)
    smax = jnp.where(jnp.isfinite(smax), smax, 0.0)
    smax_e = smax[edge_dst]
    ex, wv = _edge_exp_weight(score, smax_e, v_e)
    denom = jax.ops.segment_sum(ex, edge_dst, num_segments=_N)
    agg = jax.ops.segment_sum(wv, edge_dst, num_segments=_N)
    aggn = _agg_normalize(agg, denom)
    x = _mm(aggn, p[pre + "Wo"], residual=x)

    h2 = _ln(x, p[pre + "ln2_g"], p[pre + "ln2_b"])
    t = _mm(h2, p[pre + "ffn_w1"], p[pre + "ffn_b1"], act="gelu")
    x = _mm(t, p[pre + "ffn_w2"], p[pre + "ffn_b2"], residual=x)

  return _ln(x, p["norm_g"], p["norm_b"])


@jax.jit
def kernel(atomic_numbers, pos, batch, edge_src, edge_dst, params):
  return _forward_impl(atomic_numbers, pos, batch, edge_src, edge_dst, params)


# speedup vs baseline: 3.6648x; 1.0083x over previous
import functools

import numpy as np
import jax
import jax.numpy as jnp
from jax import lax
from jax.experimental import pallas as pl
from jax.experimental.pallas import tpu as pltpu
from jax.experimental.pallas import tpu_sc as plsc

_N = 10000
_E = 160000
_D = 480
_NB = 128
_NH = 4
_HD = _D // _NH
_MAXR = 5.0
_AVGDEG = 15.57930850982666
_NW = 32

_F32 = jnp.float32

_SEL = np.zeros((_D, 8), np.float32)
for _c in range(_D):
  _SEL[_c, _c // _HD] = 1.0
_SELT = np.ascontiguousarray(_SEL.T)


def _pad_rows(a, m):
  r = (-a.shape[0]) % m
  if r == 0:
    return a
  return jnp.concatenate([a, jnp.zeros((r,) + a.shape[1:], a.dtype)], axis=0)


def _pad2(a, rows, cols):
  return jnp.pad(a, ((0, rows - a.shape[0]), (0, cols - a.shape[1])))


def _sc_gather_call(table, idx, chunk):
  v_rows, d = table.shape
  b = idx.shape[0]
  bpw = b // _NW
  nch = bpw // chunk
  mesh = plsc.VectorSubcoreMesh(core_axis_name="c", subcore_axis_name="s")

  @functools.partial(
      pl.kernel,
      mesh=mesh,
      compiler_params=pltpu.CompilerParams(use_tc_tiling_on_sc=False),
      out_type=jax.ShapeDtypeStruct((b, d), _F32),
      scratch_types=[
          pltpu.VMEM((chunk,), jnp.int32),
          pltpu.VMEM((chunk,), jnp.int32),
          pltpu.VMEM((chunk, d), _F32),
          pltpu.VMEM((chunk, d), _F32),
          pltpu.SemaphoreType.DMA,
          pltpu.SemaphoreType.DMA,
      ],
  )
  def gk(table_hbm, idx_hbm, out_hbm, idx0, idx1, rows0, rows1, sem0, sem1):
    wid = lax.axis_index("s") * 2 + lax.axis_index("c")
    w0 = wid * bpw

    def body(j2, carry):
      b0 = w0 + (2 * j2) * chunk
      b1 = b0 + chunk
      pltpu.sync_copy(idx_hbm.at[pl.ds(b0, chunk)], idx0)
      cp0 = pltpu.async_copy(table_hbm.at[idx0], rows0, sem0)
      pltpu.sync_copy(idx_hbm.at[pl.ds(b1, chunk)], idx1)
      cp1 = pltpu.async_copy(table_hbm.at[idx1], rows1, sem1)
      cp0.wait()
      wr0 = pltpu.async_copy(rows0, out_hbm.at[pl.ds(b0, chunk)], sem0)
      cp1.wait()
      wr1 = pltpu.async_copy(rows1, out_hbm.at[pl.ds(b1, chunk)], sem1)
      wr0.wait()
      wr1.wait()
      return carry

    lax.fori_loop(0, nch // 2, body, 0)
    if nch % 2:
      bl = w0 + (nch - 1) * chunk
      pltpu.sync_copy(idx_hbm.at[pl.ds(bl, chunk)], idx0)
      pltpu.async_copy(table_hbm.at[idx0], rows0, sem0).wait()
      pltpu.sync_copy(rows0, out_hbm.at[pl.ds(bl, chunk)])

  return gk(table, idx)


def _gather_rows(table, idx, chunk=128):
  b0 = idx.shape[0]
  idxp = _pad_rows(idx.astype(jnp.int32), _NW * chunk)
  out = _sc_gather_call(table, idxp, chunk)
  return out[:b0]


def _mm(x, w, b=None, act=None, mul=None, residual=None):
  m, k = x.shape
  _, n = w.shape
  bm = 2000 if m % 2000 == 0 else 1000
  grid = m // bm

  has_b = b is not None
  has_mul = mul is not None
  has_res = residual is not None

  def body(*refs):
    i = 0
    x_ref = refs[i]; i += 1
    w_ref = refs[i]; i += 1
    acc = jnp.dot(x_ref[...], w_ref[...], preferred_element_type=_F32)
    if has_b:
      acc = acc + refs[i][...]; i += 1
    if act == "silu":
      acc = acc * jax.nn.sigmoid(acc)
    elif act == "gelu":
      acc = jax.nn.gelu(acc)
    if has_mul:
      acc = acc * refs[i][...]; i += 1
    if has_res:
      acc = acc + refs[i][...]; i += 1
    refs[i][...] = acc

  in_specs = [
      pl.BlockSpec((bm, k), lambda i: (i, 0)),
      pl.BlockSpec((k, n), lambda i: (0, 0)),
  ]
  args = [x, w]
  if has_b:
    in_specs.append(pl.BlockSpec((1, n), lambda i: (0, 0)))
    args.append(b.reshape(1, n))
  if has_mul:
    in_specs.append(pl.BlockSpec((bm, n), lambda i: (i, 0)))
    args.append(mul)
  if has_res:
    in_specs.append(pl.BlockSpec((bm, n), lambda i: (i, 0)))
    args.append(residual)

  return pl.pallas_call(
      body,
      grid=(grid,),
      in_specs=in_specs,
      out_specs=pl.BlockSpec((bm, n), lambda i: (i, 0)),
      out_shape=jax.ShapeDtypeStruct((m, n), _F32),
  )(*args)


def _ln(x, g, b):
  m, d = x.shape
  bm = 2000 if m % 2000 == 0 else 1000

  def body(x_ref, g_ref, b_ref, o_ref):
    xv = x_ref[...]
    mu = jnp.mean(xv, axis=1, keepdims=True)
    xc = xv - mu
    var = jnp.mean(xc * xc, axis=1, keepdims=True)
    o_ref[...] = xc * lax.rsqrt(var + 1e-5) * g_ref[...] + b_ref[...]

  return pl.pallas_call(
      body,
      grid=(m // bm,),
      in_specs=[
          pl.BlockSpec((bm, d), lambda i: (i, 0)),
          pl.BlockSpec((1, d), lambda i: (0, 0)),
          pl.BlockSpec((1, d), lambda i: (0, 0)),
      ],
      out_specs=pl.BlockSpec((bm, d), lambda i: (i, 0)),
      out_shape=jax.ShapeDtypeStruct((m, d), _F32),
  )(x, g.reshape(1, d), b.reshape(1, d))


def _edge_features(pos_src, pos_dst):
  e = pos_src.shape[0]
  bm = 2000

  def body(ps_ref, pd_ref, sh_ref, rbf_ref):
    dv = ps_ref[...] - pd_ref[...]
    d2 = jnp.sum(dv * dv, axis=1, keepdims=True)
    dlen = jnp.sqrt(d2)
    inv = 1.0 / jnp.maximum(dlen, 1e-9)
    lane = lax.broadcasted_iota(jnp.int32, (bm, 16), 1)
    xc = jnp.sum(jnp.where(lane == 0, dv, 0.0), axis=1, keepdims=True) * inv
    yc = jnp.sum(jnp.where(lane == 1, dv, 0.0), axis=1, keepdims=True) * inv
    zc = jnp.sum(jnp.where(lane == 2, dv, 0.0), axis=1, keepdims=True) * inv
    s3 = np.sqrt(3.0).astype(np.float32)
    s15 = np.sqrt(15.0).astype(np.float32)
    s5 = np.sqrt(5.0).astype(np.float32)
    vals = [
        jnp.ones_like(xc),
        s3 * xc, s3 * yc, s3 * zc,
        s15 * xc * yc, s15 * yc * zc,
        0.5 * s5 * (3.0 * zc * zc - 1.0),
        s15 * xc * zc,
        0.5 * s15 * (xc * xc - yc * yc),
    ]
    sh = jnp.zeros((bm, 16), _F32)
    for i, v in enumerate(vals):
      sh = jnp.where(lane == i, v, sh)
    sh_ref[...] = sh
    centers = lax.broadcasted_iota(jnp.int32, (bm, _NB), 1).astype(_F32) * (
        _MAXR / (_NB - 1.0))
    z = (dlen - centers) * (_NB / _MAXR)
    rbf_ref[...] = jnp.exp(-0.5 * z * z)

  return pl.pallas_call(
      body,
      grid=(e // bm,),
      in_specs=[
          pl.BlockSpec((bm, 16), lambda i: (i, 0)),
          pl.BlockSpec((bm, 16), lambda i: (i, 0)),
      ],
      out_specs=[
          pl.BlockSpec((bm, 16), lambda i: (i, 0)),
          pl.BlockSpec((bm, _NB), lambda i: (i, 0)),
      ],
      out_shape=[
          jax.ShapeDtypeStruct((e, 16), _F32),
          jax.ShapeDtypeStruct((e, _NB), _F32),
      ],
  )(pos_src, pos_dst)


def _edge_scores(q_e, k_e, gate8, extra8):
  e = q_e.shape[0]
  bm = 2000
  scale = 1.0 / np.sqrt(_HD)

  def body(q_ref, k_ref, g_ref, x_ref, s_ref, o_ref):
    prod = q_ref[...] * k_ref[...]
    s = jnp.dot(prod, s_ref[...], preferred_element_type=_F32) * scale
    o_ref[...] = s * g_ref[...] + x_ref[...]

  return pl.pallas_call(
      body,
      grid=(e // bm,),
      in_specs=[
          pl.BlockSpec((bm, _D), lambda i: (i, 0)),
          pl.BlockSpec((bm, _D), lambda i: (i, 0)),
          pl.BlockSpec((bm, 8), lambda i: (i, 0)),
          pl.BlockSpec((bm, 8), lambda i: (i, 0)),
          pl.BlockSpec((_D, 8), lambda i: (0, 0)),
      ],
      out_specs=pl.BlockSpec((bm, 8), lambda i: (i, 0)),
      out_shape=jax.ShapeDtypeStruct((e, 8), _F32),
  )(q_e, k_e, gate8, extra8, _SEL)


def _edge_exp_weight(score, smax_e, v_e):
  e = score.shape[0]
  bm = 2000

  def body(s_ref, m_ref, v_ref, t_ref, ex_ref, wv_ref):
    ex = jnp.exp(s_ref[...] - m_ref[...])
    ex_ref[...] = ex
    exb = jnp.dot(ex, t_ref[...], preferred_element_type=_F32)
    wv_ref[...] = v_ref[...] * exb

  return pl.pallas_call(
      body,
      grid=(e // bm,),
      in_specs=[
          pl.BlockSpec((bm, 8), lambda i: (i, 0)),
          pl.BlockSpec((bm, 8), lambda i: (i, 0)),
          pl.BlockSpec((bm, _D), lambda i: (i, 0)),
          pl.BlockSpec((8, _D), lambda i: (0, 0)),
      ],
      out_specs=[
          pl.BlockSpec((bm, 8), lambda i: (i, 0)),
          pl.BlockSpec((bm, _D), lambda i: (i, 0)),
      ],
      out_shape=[
          jax.ShapeDtypeStruct((e, 8), _F32),
          jax.ShapeDtypeStruct((e, _D), _F32),
      ],
  )(score, smax_e, v_e, _SELT)


def _agg_normalize(agg, denom):
  m = agg.shape[0]
  bm = 2000 if m % 2000 == 0 else 1000

  def body(a_ref, d_ref, t_ref, o_ref):
    rcp = 1.0 / (d_ref[...] + 1e-9)
    o_ref[...] = a_ref[...] * jnp.dot(rcp, t_ref[...],
                                      preferred_element_type=_F32)

  return pl.pallas_call(
      body,
      grid=(m // bm,),
      in_specs=[
          pl.BlockSpec((bm, _D), lambda i: (i, 0)),
          pl.BlockSpec((bm, 8), lambda i: (i, 0)),
          pl.BlockSpec((8, _D), lambda i: (0, 0)),
      ],
      out_specs=pl.BlockSpec((bm, _D), lambda i: (i, 0)),
      out_shape=jax.ShapeDtypeStruct((m, _D), _F32),
  )(agg, denom, _SELT)


def _forward_impl(atomic_numbers, pos, batch, edge_src, edge_dst, params):
  p = params
  del batch

  pos16 = jnp.pad(pos, ((0, 0), (0, 13)))
  ps = _gather_rows(pos16, edge_src, chunk=512)
  pd = _gather_rows(pos16, edge_dst, chunk=512)
  sh16, rbf = _edge_features(ps, pd)

  atom_emb = _gather_rows(p["atom_embed"], atomic_numbers, chunk=128)

  r1 = _mm(rbf, p["deg_w1"], p["deg_b1"], act="silu")
  r2 = _mm(r1, p["deg_w2"], p["deg_b2"], act="silu")
  w3p = _pad2(p["deg_w3"], 64, 16)
  b3p = jnp.pad(p["deg_b3"], (0, 7))
  msg16 = _mm(r2, w3p, b3p, mul=sh16)
  deg16 = jax.ops.segment_sum(msg16, edge_dst, num_segments=_N)
  projp = _pad2(p["deg_proj"], 16, _D) * np.float32(1.0 / np.sqrt(_AVGDEG))
  x = _mm(deg16, projp, residual=atom_emb)

  for i in range(4):
    pre = "blk%d_" % i
    h = _ln(x, p[pre + "ln1_g"], p[pre + "ln1_b"])
    qt = _mm(h, p[pre + "Wq"])
    kt = _mm(h, p[pre + "Wk"])
    vt = _mm(h, p[pre + "Wv"])
    q_e = _gather_rows(qt, edge_dst, chunk=128)
    k_e = _gather_rows(kt, edge_src, chunk=128)
    v_e = _gather_rows(vt, edge_src, chunk=128)

    g1 = _mm(rbf, p[pre + "gate_w1"], p[pre + "gate_b1"], act="silu")
    g2 = _mm(g1, p[pre + "gate_w2"], p[pre + "gate_b2"], act="silu")
    gw3p = _pad2(p[pre + "gate_w3"], 64, 8)
    gb3p = jnp.pad(p[pre + "gate_b3"], (0, 4))
    gate8 = _mm(g2, gw3p, gb3p)
    mixp = _pad2(p[pre + "sh_mix"], 16, 8)
    extra8 = _mm(sh16, mixp)

    score = _edge_scores(q_e, k_e, gate8, extra8)
    smax = jax.ops.segment_max(score, edge_dst, num_segments=_N)
    smax = jnp.where(jnp.isfinite(smax), smax, 0.0)
    smax_e = smax[edge_dst]
    ex, wv = _edge_exp_weight(score, smax_e, v_e)
    denom = jax.ops.segment_sum(ex, edge_dst, num_segments=_N)
    agg = jax.ops.segment_sum(wv, edge_dst, num_segments=_N)
    aggn = _agg_normalize(agg, denom)
    x = _mm(aggn, p[pre + "Wo"], residual=x)

    h2 = _ln(x, p[pre + "ln2_g"], p[pre + "ln2_b"])
    t = _mm(h2, p[pre + "ffn_w1"], p[pre + "ffn_b1"], act="gelu")
    x = _mm(t, p[pre + "ffn_w2"], p[pre + "ffn_b2"], residual=x)

  return _ln(x, p["norm_g"], p["norm_b"])


@jax.jit
def kernel(atomic_numbers, pos, batch, edge_src, edge_dst, params):
  return _forward_impl(atomic_numbers, pos, batch, edge_src, edge_dst, params)
